# baseline (speedup 1.0000x reference)
import functools

import jax
import jax.numpy as jnp
from jax import lax
from jax.experimental import pallas as pl
from jax.experimental.pallas import tpu as pltpu
from jax.experimental.pallas import tpu_sc as plsc

D = 768
T = 77
TP = 80
NA = 32
NBR = 32
NCR = 16
TA = 72
B = 1024
NC, NS = 2, 16
NW = NC * NS
BPW = B // NW


def _sc_gather(rec, table):
    mesh = plsc.VectorSubcoreMesh(core_axis_name="c", subcore_axis_name="s")

    @functools.partial(
        pl.kernel,
        mesh=mesh,
        out_type=(
            jax.ShapeDtypeStruct((B, T, D), jnp.float32),
            jax.ShapeDtypeStruct((B, TP - TA, D), jnp.float32),
        ),
        scratch_types=[
            pltpu.VMEM((BPW * TP,), jnp.int32),
            pltpu.VMEM((2, NA, D), jnp.float32),
            pltpu.VMEM((2, NBR, D), jnp.float32),
            pltpu.VMEM((2, NCR, D), jnp.float32),
            pltpu.SemaphoreType.DMA,
            pltpu.SemaphoreType.DMA,
            pltpu.SemaphoreType.DMA,
            pltpu.SemaphoreType.DMA,
            pltpu.SemaphoreType.DMA,
            pltpu.SemaphoreType.DMA,
            pltpu.SemaphoreType.DMA,
            pltpu.SemaphoreType.DMA,
            pltpu.SemaphoreType.DMA,
            pltpu.SemaphoreType.DMA,
            pltpu.SemaphoreType.DMA,
            pltpu.SemaphoreType.DMA,
        ],
    )
    def k(rec_hbm, table_hbm, out_hbm, side_hbm, idx_v, bufA, bufB, bufC,
          ga0, ga1, gb0, gb1, gc0, gc1, oa0, oa1, ob0, ob1, oc0, oc1):
        wid = lax.axis_index("s") * NC + lax.axis_index("c")
        b0 = wid * BPW
        pltpu.sync_copy(rec_hbm.at[wid], idx_v)
        ga = (ga0, ga1)
        gb = (gb0, gb1)
        gc = (gc0, gc1)
        oa = (oa0, oa1)
        ob = (ob0, ob1)
        oc = (oc0, oc1)

        def gA_start(bl, k_):
            pltpu.async_copy(
                table_hbm.at[idx_v.at[pl.ds(TP * bl, NA)]], bufA.at[k_], ga[k_])

        def gB_start(bl, k_):
            pltpu.async_copy(
                table_hbm.at[idx_v.at[pl.ds(TP * bl + NA, NBR)]], bufB.at[k_], gb[k_])

        def gC_start(bl, k_):
            pltpu.async_copy(
                table_hbm.at[idx_v.at[pl.ds(TP * bl + NA + NBR, NCR)]], bufC.at[k_], gc[k_])

        def gA_wait(k_):
            pltpu.make_async_copy(
                table_hbm.at[idx_v.at[pl.ds(0, NA)]], bufA.at[k_], ga[k_]).wait()

        def gB_wait(k_):
            pltpu.make_async_copy(
                table_hbm.at[idx_v.at[pl.ds(0, NBR)]], bufB.at[k_], gb[k_]).wait()

        def gC_wait(k_):
            pltpu.make_async_copy(
                table_hbm.at[idx_v.at[pl.ds(0, NCR)]], bufC.at[k_], gc[k_]).wait()

        def oA_start(bl, k_):
            pltpu.async_copy(
                bufA.at[k_], out_hbm.at[b0 + bl, pl.ds(0, NA)], oa[k_])

        def oB_start(bl, k_):
            pltpu.async_copy(
                bufB.at[k_], out_hbm.at[b0 + bl, pl.ds(NA, NBR)], ob[k_])

        def oC_start(bl, k_):
            pltpu.async_copy(
                bufC.at[k_, pl.ds(0, TA - NA - NBR)],
                out_hbm.at[b0 + bl, pl.ds(NA + NBR, TA - NA - NBR)], oc[k_])
            pltpu.async_copy(
                bufC.at[k_, pl.ds(TA - NA - NBR, TP - TA)], side_hbm.at[b0 + bl], oc[k_])

        def oA_wait(k_):
            pltpu.make_async_copy(
                bufA.at[k_], out_hbm.at[b0, pl.ds(0, NA)], oa[k_]).wait()

        def oB_wait(k_):
            pltpu.make_async_copy(
                bufB.at[k_], out_hbm.at[b0, pl.ds(NA, NBR)], ob[k_]).wait()

        def oC_wait(k_):
            pltpu.make_async_copy(
                bufC.at[k_, pl.ds(0, TA - NA - NBR)],
                out_hbm.at[b0, pl.ds(NA + NBR, TA - NA - NBR)], oc[k_]).wait()
            pltpu.make_async_copy(
                bufC.at[k_, pl.ds(TA - NA - NBR, TP - TA)], side_hbm.at[b0], oc[k_]).wait()

        gA_start(0, 0); gB_start(0, 0); gC_start(0, 0)
        gA_start(1, 1); gB_start(1, 1); gC_start(1, 1)

        def body(i, carry):
            bl = 2 * i
            gA_wait(0); oA_start(bl, 0)
            gB_wait(0); oB_start(bl, 0)
            gC_wait(0); oC_start(bl, 0)
            gA_wait(1); oA_start(bl + 1, 1)
            gB_wait(1); oB_start(bl + 1, 1)
            gC_wait(1); oC_start(bl + 1, 1)
            oA_wait(0); gA_start(bl + 2, 0)
            oB_wait(0); gB_start(bl + 2, 0)
            oC_wait(0); gC_start(bl + 2, 0)
            oA_wait(1); gA_start(bl + 3, 1)
            oB_wait(1); gB_start(bl + 3, 1)
            oC_wait(1); gC_start(bl + 3, 1)
            return carry

        lax.fori_loop(0, BPW // 2 - 1, body, 0)
        gA_wait(0); oA_start(BPW - 2, 0)
        gB_wait(0); oB_start(BPW - 2, 0)
        gC_wait(0); oC_start(BPW - 2, 0)
        gA_wait(1); oA_start(BPW - 1, 1)
        gB_wait(1); oB_start(BPW - 1, 1)
        gC_wait(1); oC_start(BPW - 1, 1)
        oA_wait(0); oB_wait(0); oC_wait(0)
        oA_wait(1); oB_wait(1); oC_wait(1)

    return k(rec, table)


def kernel(tokens, token_embeddings, positional_embeddings):
    tok = tokens.astype(jnp.int32)
    rec = jnp.pad(tok, ((0, 0), (0, TP - T)))
    rec = rec.reshape(NW, BPW * TP)
    main, side = _sc_gather(rec, token_embeddings)
    out = lax.dynamic_update_slice(main, side[:, : T - TA, :], (0, TA, 0))
    return lax.cond(
        jnp.any(positional_embeddings != 0.0),
        lambda a: a + positional_embeddings[None, :, :],
        lambda a: a,
        out,
    )
